# baseline (speedup 1.0000x reference)
import jax
import jax.numpy as jnp
from jax import lax
from jax.experimental import pallas as pl
from jax.experimental.pallas import tpu as pltpu
from jax.experimental.pallas import tpu_sc as plsc

BATCH = 4096
NUM_POS = 77
EMBED_DIM = 768
LANES = 16
PAIRS = EMBED_DIM // 32
ROWW = EMBED_DIM // 2
NUM_CORES = 2
NUM_WORKERS = 32
BE_PER_WORKER = BATCH // NUM_WORKERS
IDX_BLK = 8
WAVES = (16, 16, 16, 16, 13)
NWB = 3


def _body(idx_hbm, table_hbm, pos_hbm, out_hbm, idx_v, pos_v, buf2,
          wv0, wv1, wv2, isem, g0, g1, g2, wsem):
    waves = (wv0, wv1, wv2)
    gsem = (g0, g1, g2)
    wid = lax.axis_index("s") * NUM_CORES + lax.axis_index("c")
    base = wid * BE_PER_WORKER

    pltpu.sync_copy(pos_hbm, pos_v)

    def fire_wave(i, w):
        w0 = 16 * w
        n = WAVES[w]
        off = min(w0, NUM_POS - LANES)
        v = idx_v[i, pl.ds(off, LANES)]
        for m in range(w0 - off, w0 - off + n):
            r = m - (w0 - off)
            t = v[m]
            pltpu.async_copy(
                table_hbm.at[pl.ds(t * ROWW, ROWW)],
                waves[w % NWB].at[pl.ds(r * ROWW, ROWW)],
                gsem[w % NWB])

    def decode(ref, word_off):
        q = plsc.bitcast(ref[pl.ds(word_off, LANES)], jnp.int32)
        lo = plsc.bitcast(q << 16, jnp.float32)
        hi = plsc.bitcast(q & jnp.int32(-65536), jnp.float32)
        return lo, hi

    def drain_assemble_wave(w):
        w0 = 16 * w
        n = WAVES[w]
        wb = waves[w % NWB]
        sem = gsem[w % NWB]

        def row_step(jj, _):
            pltpu.make_async_copy(
                table_hbm.at[pl.ds(0, ROWW)],
                wb.at[pl.ds(jj * ROWW, ROWW)], sem).wait()
            j = w0 + jj
            for c in range(PAIRS):
                tlo, thi = decode(wb, jj * ROWW + c * LANES)
                plo, phi = decode(pos_v, j * ROWW + c * LANES)
                buf2[j, pl.ds(c * 32, LANES)] = tlo + plo
                buf2[j, pl.ds(c * 32 + LANES, LANES)] = thi + phi
            return 0

        lax.fori_loop(0, n, row_step, 0, unroll=False)

    def blk_step(k, _):
        pltpu.async_copy(idx_hbm.at[pl.ds(base + k * IDX_BLK, IDX_BLK)],
                         idx_v, isem).wait()

        def elem_step(i, _):
            e = k * IDX_BLK + i
            fire_wave(i, 0)
            fire_wave(i, 1)
            @pl.when(jnp.logical_or(k > 0, i > 0))
            def _():
                pltpu.make_async_copy(buf2, out_hbm.at[base], wsem).wait()
            for w in range(len(WAVES)):
                if w + 2 < len(WAVES):
                    fire_wave(i, w + 2)
                drain_assemble_wave(w)
            pltpu.async_copy(buf2, out_hbm.at[base + e], wsem)
            return 0

        lax.fori_loop(0, IDX_BLK, elem_step, 0, unroll=False)
        return 0

    lax.fori_loop(0, BE_PER_WORKER // IDX_BLK, blk_step, 0, unroll=False)
    pltpu.make_async_copy(buf2, out_hbm.at[base], wsem).wait()


@jax.jit
def _embed(idx, table_pairs, pos_pairs):
    mesh = plsc.VectorSubcoreMesh(core_axis_name="c", subcore_axis_name="s")
    fn = pl.kernel(
        _body,
        out_type=jax.ShapeDtypeStruct((BATCH, NUM_POS, EMBED_DIM), jnp.float32),
        mesh=mesh,
        compiler_params=pltpu.CompilerParams(use_tc_tiling_on_sc=True,
                                             needs_layout_passes=False),
        scratch_types=[
            pltpu.VMEM((IDX_BLK, NUM_POS), jnp.int32),
            pltpu.VMEM((NUM_POS * ROWW,), jnp.float32),
            pltpu.VMEM((NUM_POS, EMBED_DIM), jnp.float32),
            pltpu.VMEM((LANES * ROWW,), jnp.float32),
            pltpu.VMEM((LANES * ROWW,), jnp.float32),
            pltpu.VMEM((LANES * ROWW,), jnp.float32),
            pltpu.SemaphoreType.DMA,
            pltpu.SemaphoreType.DMA,
            pltpu.SemaphoreType.DMA,
            pltpu.SemaphoreType.DMA,
            pltpu.SemaphoreType.DMA,
        ],
    )
    return fn(idx, table_pairs, pos_pairs)


def _pack_pairs(arr2d, rows):
    sh = (arr2d.reshape(rows, PAIRS, 2, LANES)
          .transpose(0, 1, 3, 2)
          .reshape(rows * EMBED_DIM)
          .astype(jnp.bfloat16))
    return lax.bitcast_convert_type(
        sh.reshape(rows * ROWW, 2), jnp.float32)


def kernel(input_tokens, token_table, pos_table):
    idx = input_tokens.astype(jnp.int32)
    table_pairs = _pack_pairs(token_table, token_table.shape[0])
    pos_pairs = _pack_pairs(pos_table, NUM_POS)
    return _embed(idx, table_pairs, pos_pairs)


# speedup vs baseline: 1.7981x; 1.7981x over previous
import jax
import jax.numpy as jnp
from jax import lax
from jax.experimental import pallas as pl
from jax.experimental.pallas import tpu as pltpu
from jax.experimental.pallas import tpu_sc as plsc

BATCH = 4096
NUM_POS = 77
EMBED_DIM = 768
LANES = 16
PAIRS = EMBED_DIM // 32
NUM_CORES = 2
NUM_WORKERS = 32
BE_PER_WORKER = BATCH // NUM_WORKERS
IDX_BLK = 8
WAVES = (16, 16, 16, 16, 13)


def _body(idx_hbm, table_hbm, pos_hbm, out_hbm, idx_v, pos_v, buf2, wv0, wv1,
          isem, g0, g1, wsem):
    waves = (wv0, wv1)
    gsem = (g0, g1)
    wid = lax.axis_index("s") * NUM_CORES + lax.axis_index("c")
    base = wid * BE_PER_WORKER

    pltpu.sync_copy(pos_hbm, pos_v)

    def fire_wave(i, w):
        w0 = 16 * w
        n = WAVES[w]
        off = min(w0, NUM_POS - LANES)
        v = idx_v[i, pl.ds(off, LANES)]
        for m in range(w0 - off, w0 - off + n):
            r = m - (w0 - off)
            t = v[m]
            pltpu.async_copy(
                table_hbm.at[pl.ds(t * EMBED_DIM, EMBED_DIM)],
                waves[w % 2].at[pl.ds(r * EMBED_DIM, EMBED_DIM)],
                gsem[w % 2])

    def drain_assemble_wave(w):
        w0 = 16 * w
        n = WAVES[w]
        wb = waves[w % 2]
        sem = gsem[w % 2]

        @plsc.parallel_loop(0, n)
        def row_step(jj):
            pltpu.make_async_copy(
                table_hbm.at[pl.ds(0, EMBED_DIM)],
                wb.at[pl.ds(jj * EMBED_DIM, EMBED_DIM)], sem).wait()
            j = w0 + jj
            for c in range(PAIRS):
                q = plsc.bitcast(
                    pos_v[pl.ds(j * (EMBED_DIM // 2) + c * LANES, LANES)],
                    jnp.int32)
                lo = plsc.bitcast(q << 16, jnp.float32)
                hi = plsc.bitcast(q & jnp.int32(-65536), jnp.float32)
                ta = wb[pl.ds(jj * EMBED_DIM + c * 32, LANES)]
                tb = wb[pl.ds(jj * EMBED_DIM + c * 32 + LANES, LANES)]
                buf2[j, pl.ds(c * 32, LANES)] = ta + lo
                buf2[j, pl.ds(c * 32 + LANES, LANES)] = tb + hi

    def blk_step(k, _):
        pltpu.async_copy(idx_hbm.at[pl.ds(base + k * IDX_BLK, IDX_BLK)],
                         idx_v, isem).wait()

        def elem_step(i, _):
            e = k * IDX_BLK + i
            fire_wave(i, 0)
            fire_wave(i, 1)
            @pl.when(jnp.logical_or(k > 0, i > 0))
            def _():
                pltpu.make_async_copy(buf2, out_hbm.at[base], wsem).wait()
            for w in range(len(WAVES)):
                drain_assemble_wave(w)
                if w + 2 < len(WAVES):
                    fire_wave(i, w + 2)
            pltpu.async_copy(buf2, out_hbm.at[base + e], wsem)
            return 0

        lax.fori_loop(0, IDX_BLK, elem_step, 0, unroll=False)
        return 0

    lax.fori_loop(0, BE_PER_WORKER // IDX_BLK, blk_step, 0, unroll=False)
    pltpu.make_async_copy(buf2, out_hbm.at[base], wsem).wait()


@jax.jit
def _embed(idx, table_flat, pos_sh):
    mesh = plsc.VectorSubcoreMesh(core_axis_name="c", subcore_axis_name="s")
    fn = pl.kernel(
        _body,
        out_type=jax.ShapeDtypeStruct((BATCH, NUM_POS, EMBED_DIM), jnp.float32),
        mesh=mesh,
        compiler_params=pltpu.CompilerParams(use_tc_tiling_on_sc=True,
                                             needs_layout_passes=False),
        scratch_types=[
            pltpu.VMEM((IDX_BLK, NUM_POS), jnp.int32),
            pltpu.VMEM((NUM_POS * EMBED_DIM,), jnp.bfloat16),
            pltpu.VMEM((NUM_POS, EMBED_DIM), jnp.float32),
            pltpu.VMEM((LANES * EMBED_DIM,), jnp.float32),
            pltpu.VMEM((LANES * EMBED_DIM,), jnp.float32),
            pltpu.SemaphoreType.DMA,
            pltpu.SemaphoreType.DMA,
            pltpu.SemaphoreType.DMA,
            pltpu.SemaphoreType.DMA,
        ],
    )
    return fn(idx, table_flat, pos_sh)


def kernel(input_tokens, token_table, pos_table):
    idx = input_tokens.astype(jnp.int32)
    table_flat = token_table.reshape(-1)
    pos_sh = (pos_table.reshape(NUM_POS, PAIRS, 2, LANES)
              .transpose(0, 1, 3, 2)
              .reshape(NUM_POS * EMBED_DIM)
              .astype(jnp.bfloat16))
    pos_pairs = lax.bitcast_convert_type(
        pos_sh.reshape(NUM_POS * EMBED_DIM // 2, 2), jnp.float32)
    return _embed(idx, table_flat, pos_pairs)


# speedup vs baseline: 2.8719x; 1.5973x over previous
import jax
import jax.numpy as jnp
from jax import lax
from jax.experimental import pallas as pl
from jax.experimental.pallas import tpu as pltpu
from jax.experimental.pallas import tpu_sc as plsc

BATCH = 4096
NUM_POS = 77
EMBED_DIM = 768
LANES = 16
PAIRS = EMBED_DIM // 32
ROWW = EMBED_DIM // 2
NUM_CORES = 2
NUM_WORKERS = 32
BE_PER_WORKER = BATCH // NUM_WORKERS
NGRP = BE_PER_WORKER // 8


def _body(idxT_hbm, table_hbm, pos_hbm, out_hbm, idx_v, pos_v,
          ob0, ob1, wv0, wv1, isem, g0, g1, u0, u1):
    obufs = (ob0, ob1)
    waves = (wv0, wv1)
    gsem = (g0, g1)
    usem = (u0, u1)
    wid = lax.axis_index("s") * NUM_CORES + lax.axis_index("c")
    base = wid * BE_PER_WORKER

    pltpu.sync_copy(pos_hbm, pos_v)
    pltpu.async_copy(idxT_hbm.at[slice(None), pl.ds(base, BE_PER_WORKER)],
                     idx_v, isem).wait()

    def group(kp, g):
        k = kp * 2 + g
        col16 = kp * LANES
        bcol = g * 8

        def fire(p, wb):
            v = idx_v[p, pl.ds(col16, LANES)]
            for i in range(8):
                t = v[bcol + i]
                pltpu.async_copy(
                    table_hbm.at[pl.ds(t * EMBED_DIM, EMBED_DIM)],
                    waves[wb].at[pl.ds(i * EMBED_DIM, EMBED_DIM)],
                    gsem[wb])

        def wait_write(b):
            pltpu.make_async_copy(obufs[b], out_hbm.at[0, pl.ds(base, 8)],
                                  usem[b]).wait()

        def assemble_write(p, b):
            wb = waves[b]
            for i in range(8):
                pltpu.make_async_copy(
                    table_hbm.at[pl.ds(0, EMBED_DIM)],
                    wb.at[pl.ds(i * EMBED_DIM, EMBED_DIM)], gsem[b]).wait()
            for c in range(PAIRS):
                q = plsc.bitcast(pos_v[pl.ds(p * ROWW + c * LANES, LANES)],
                                 jnp.int32)
                lo = plsc.bitcast(q << 16, jnp.float32)
                hi = plsc.bitcast(q & jnp.int32(-65536), jnp.float32)
                for i in range(8):
                    ta = wb[pl.ds(i * EMBED_DIM + c * 32, LANES)]
                    tb = wb[pl.ds(i * EMBED_DIM + c * 32 + LANES, LANES)]
                    obufs[b][i, pl.ds(c * 32, LANES)] = ta + lo
                    obufs[b][i, pl.ds(c * 32 + LANES, LANES)] = tb + hi
            pltpu.async_copy(obufs[b],
                             out_hbm.at[p, pl.ds(base + k * 8, 8)], usem[b])

        fire(0, 0)

        def pos_step(p, _):
            @pl.when(lax.rem(p, 2) == 0)
            def _():
                @pl.when(p < NUM_POS - 1)
                def _():
                    fire(p + 1, 1)
                if g == 0:
                    @pl.when(jnp.logical_or(kp > 0, p >= 2))
                    def _():
                        wait_write(0)
                else:
                    wait_write(0)
                assemble_write(p, 0)

            @pl.when(lax.rem(p, 2) == 1)
            def _():
                @pl.when(p < NUM_POS - 1)
                def _():
                    fire(p + 1, 0)
                if g == 0:
                    @pl.when(jnp.logical_or(kp > 0, p >= 2))
                    def _():
                        wait_write(1)
                else:
                    wait_write(1)
                assemble_write(p, 1)

            return 0

        lax.fori_loop(0, NUM_POS, pos_step, 0, unroll=False)

    def kp_step(kp, _):
        group(kp, 0)
        group(kp, 1)
        return 0

    lax.fori_loop(0, NGRP // 2, kp_step, 0, unroll=False)
    wait_write0 = pltpu.make_async_copy(ob0, out_hbm.at[0, pl.ds(base, 8)], u0)
    wait_write0.wait()
    pltpu.make_async_copy(ob1, out_hbm.at[0, pl.ds(base, 8)], u1).wait()


@jax.jit
def _embed(idxT, table_flat, pos_pairs):
    mesh = plsc.VectorSubcoreMesh(core_axis_name="c", subcore_axis_name="s")
    fn = pl.kernel(
        _body,
        out_type=jax.ShapeDtypeStruct((NUM_POS, BATCH, EMBED_DIM), jnp.float32),
        mesh=mesh,
        compiler_params=pltpu.CompilerParams(use_tc_tiling_on_sc=True,
                                             needs_layout_passes=False),
        scratch_types=[
            pltpu.VMEM((NUM_POS, BE_PER_WORKER), jnp.int32),
            pltpu.VMEM((NUM_POS * ROWW,), jnp.float32),
            pltpu.VMEM((8, EMBED_DIM), jnp.float32),
            pltpu.VMEM((8, EMBED_DIM), jnp.float32),
            pltpu.VMEM((8 * EMBED_DIM,), jnp.float32),
            pltpu.VMEM((8 * EMBED_DIM,), jnp.float32),
            pltpu.SemaphoreType.DMA,
            pltpu.SemaphoreType.DMA,
            pltpu.SemaphoreType.DMA,
            pltpu.SemaphoreType.DMA,
            pltpu.SemaphoreType.DMA,
        ],
    )
    return fn(idxT, table_flat, pos_pairs)


def kernel(input_tokens, token_table, pos_table):
    idxT = jnp.transpose(input_tokens.astype(jnp.int32))
    table_flat = token_table.reshape(-1)
    pos_sh = (pos_table.reshape(NUM_POS, PAIRS, 2, LANES)
              .transpose(0, 1, 3, 2)
              .reshape(NUM_POS * EMBED_DIM)
              .astype(jnp.bfloat16))
    pos_pairs = lax.bitcast_convert_type(
        pos_sh.reshape(NUM_POS * ROWW, 2), jnp.float32)
    out_pm = _embed(idxT, table_flat, pos_pairs)
    return jnp.transpose(out_pm, (1, 0, 2))


# speedup vs baseline: 2.8738x; 1.0006x over previous
import jax
import jax.numpy as jnp
from jax import lax
from jax.experimental import pallas as pl
from jax.experimental.pallas import tpu as pltpu
from jax.experimental.pallas import tpu_sc as plsc

BATCH = 4096
NUM_POS = 77
EMBED_DIM = 768
LANES = 16
PAIRS = EMBED_DIM // 32
ROWW = EMBED_DIM // 2
NUM_CORES = 2
NUM_WORKERS = 32
BE_PER_WORKER = BATCH // NUM_WORKERS
NGRP = BE_PER_WORKER // 8


def _body(idxT_hbm, table_hbm, pos_hbm, out_hbm, idx_v, pos_v,
          ir0, ir1, ob0, ob1, wv0, wv1, isem, g0, g1, u0, u1):
    obufs = (ob0, ob1)
    waves = (wv0, wv1)
    idxr = (ir0, ir1)
    gsem = (g0, g1)
    usem = (u0, u1)
    wid = lax.axis_index("s") * NUM_CORES + lax.axis_index("c")
    base = wid * BE_PER_WORKER

    pltpu.sync_copy(pos_hbm, pos_v)
    pltpu.async_copy(idxT_hbm.at[slice(None), pl.ds(base, BE_PER_WORKER)],
                     idx_v, isem).wait()

    lane = lax.iota(jnp.int32, LANES)
    div6 = [(s * LANES + lane) // 6 for s in range(3)]
    mod6 = [(s * LANES + lane) % 6 for s in range(3)]

    def group(kp, g):
        k = kp * 2 + g
        col16 = kp * LANES
        bcol = g * 8

        def fire(p, wb):
            v = idx_v[p, pl.ds(col16, LANES)]
            for s in range(3):
                tt = v.at[bcol + div6[s]].get(mode="promise_in_bounds")
                idxr[wb][pl.ds(s * LANES, LANES)] = tt * 6 + mod6[s]
            pltpu.async_copy(table_hbm.at[idxr[wb]], waves[wb], gsem[wb])

        def wait_write(b):
            pltpu.make_async_copy(obufs[b], out_hbm.at[0, pl.ds(base, 8)],
                                  usem[b]).wait()

        def assemble_write(p, b):
            wb = waves[b]
            pltpu.make_async_copy(table_hbm.at[idxr[b]], wb, gsem[b]).wait()
            for c in range(PAIRS):
                q = plsc.bitcast(pos_v[pl.ds(p * ROWW + c * LANES, LANES)],
                                 jnp.int32)
                lo = plsc.bitcast(q << 16, jnp.float32)
                hi = plsc.bitcast(q & jnp.int32(-65536), jnp.float32)
                for i in range(8):
                    ta = wb[i * 6 + c // 4, pl.ds((c % 4) * 32, LANES)]
                    tb = wb[i * 6 + c // 4, pl.ds((c % 4) * 32 + LANES, LANES)]
                    obufs[b][i, pl.ds(c * 32, LANES)] = ta + lo
                    obufs[b][i, pl.ds(c * 32 + LANES, LANES)] = tb + hi
            pltpu.async_copy(obufs[b],
                             out_hbm.at[p, pl.ds(base + k * 8, 8)], usem[b])

        fire(0, 0)

        def pos_step(p, _):
            @pl.when(lax.rem(p, 2) == 0)
            def _():
                @pl.when(p < NUM_POS - 1)
                def _():
                    fire(p + 1, 1)
                if g == 0:
                    @pl.when(jnp.logical_or(kp > 0, p >= 2))
                    def _():
                        wait_write(0)
                else:
                    wait_write(0)
                assemble_write(p, 0)

            @pl.when(lax.rem(p, 2) == 1)
            def _():
                @pl.when(p < NUM_POS - 1)
                def _():
                    fire(p + 1, 0)
                if g == 0:
                    @pl.when(jnp.logical_or(kp > 0, p >= 2))
                    def _():
                        wait_write(1)
                else:
                    wait_write(1)
                assemble_write(p, 1)

            return 0

        lax.fori_loop(0, NUM_POS, pos_step, 0, unroll=False)

    def kp_step(kp, _):
        group(kp, 0)
        group(kp, 1)
        return 0

    lax.fori_loop(0, NGRP // 2, kp_step, 0, unroll=False)
    wait_write0 = pltpu.make_async_copy(ob0, out_hbm.at[0, pl.ds(base, 8)], u0)
    wait_write0.wait()
    pltpu.make_async_copy(ob1, out_hbm.at[0, pl.ds(base, 8)], u1).wait()


@jax.jit
def _embed(idxT, table_flat, pos_pairs):
    mesh = plsc.VectorSubcoreMesh(core_axis_name="c", subcore_axis_name="s")
    fn = pl.kernel(
        _body,
        out_type=jax.ShapeDtypeStruct((NUM_POS, BATCH, EMBED_DIM), jnp.float32),
        mesh=mesh,
        compiler_params=pltpu.CompilerParams(use_tc_tiling_on_sc=True,
                                             needs_layout_passes=False),
        scratch_types=[
            pltpu.VMEM((NUM_POS, BE_PER_WORKER), jnp.int32),
            pltpu.VMEM((NUM_POS * ROWW,), jnp.float32),
            pltpu.VMEM((48,), jnp.int32),
            pltpu.VMEM((48,), jnp.int32),
            pltpu.VMEM((8, EMBED_DIM), jnp.float32),
            pltpu.VMEM((8, EMBED_DIM), jnp.float32),
            pltpu.VMEM((48, 128), jnp.float32),
            pltpu.VMEM((48, 128), jnp.float32),
            pltpu.SemaphoreType.DMA,
            pltpu.SemaphoreType.DMA,
            pltpu.SemaphoreType.DMA,
            pltpu.SemaphoreType.DMA,
            pltpu.SemaphoreType.DMA,
        ],
    )
    return fn(idxT, table_flat, pos_pairs)


def kernel(input_tokens, token_table, pos_table):
    idxT = jnp.transpose(input_tokens.astype(jnp.int32))
    table_flat = token_table.reshape(-1).reshape(-1, 128)
    pos_sh = (pos_table.reshape(NUM_POS, PAIRS, 2, LANES)
              .transpose(0, 1, 3, 2)
              .reshape(NUM_POS * EMBED_DIM)
              .astype(jnp.bfloat16))
    pos_pairs = lax.bitcast_convert_type(
        pos_sh.reshape(NUM_POS * ROWW, 2), jnp.float32)
    out_pm = _embed(idxT, table_flat, pos_pairs)
    return jnp.transpose(out_pm, (1, 0, 2))
